# baseline (speedup 1.0000x reference)
import functools

import jax
import jax.numpy as jnp
from jax import lax
from jax.experimental import pallas as pl
from jax.experimental.pallas import tpu as pltpu
from jax.experimental.pallas import tpu_sc as plsc

N = 50000
E = 200000
D = 128
NC = 2
NS = 16
L = 16

U = E // L
UPT32 = -(-U // (NC * NS))
UPT16 = -(-U // NS)
CHUNK = 100
BK = 3840
NBK = 14
ACC_ROWS = BK + 128
TRASH = BK
G = 128

_mesh = None


def _get_mesh():
    global _mesh
    if _mesh is None:
        _mesh = plsc.VectorSubcoreMesh(
            core_axis_name="c", subcore_axis_name="s",
            num_cores=NC, num_subcores=NS)
    return _mesh


_GDN = lax.GatherDimensionNumbers(
    offset_dims=(), collapsed_slice_dims=(0,), start_index_map=(0,))


def _vtake(x, i):
    return lax.gather(x, i[:, None], _GDN, slice_sizes=(1,),
                      mode=lax.GatherScatterMode.PROMISE_IN_BOUNDS)


def _dedup_add(cbuf, idx):
    cnt, last = plsc.scan_count(idx)
    plsc.addupdate_scatter(cbuf, [idx], cnt, mask=last)


def _deg_kernel(e0, e1, e2, dcnt, cbuf, stage):
    wid = lax.axis_index("s") * NC + lax.axis_index("c")
    lo = jnp.minimum(wid * UPT32, U)
    hi = jnp.minimum(lo + UPT32, U)
    nchunks = -(-UPT32 // CHUNK)
    for r, e in enumerate((e0, e1, e2)):
        for side in range(2):
            rs = r * 2 + side

            def zero_body(i, _):
                cbuf[pl.ds(i * L, L)] = jnp.zeros((L,), jnp.int32)
                return 0
            lax.fori_loop(0, N // L, zero_body, 0)

            def chunk_body(j, _):
                ubase = lo + j * CHUNK
                sb = jnp.minimum(ubase, U - CHUNK)
                pltpu.sync_copy(
                    e.at[pl.ds(side * E + sb * L, CHUNK * L)], stage)

                def unit_body(v, _):
                    uid = sb + v

                    @pl.when((uid >= ubase) & (uid < hi))
                    def _():
                        _dedup_add(cbuf, stage[pl.ds(v * L, L)])
                    return 0
                lax.fori_loop(0, CHUNK, unit_body, 0)
                return 0
            lax.fori_loop(0, nchunks, chunk_body, 0)
            pltpu.sync_copy(cbuf, dcnt.at[rs * (NC * NS) + wid])


def _degrees(e0, e1, e2):
    k = functools.partial(
        pl.kernel, mesh=_get_mesh(),
        compiler_params=pltpu.CompilerParams(needs_layout_passes=False),
        out_type=jax.ShapeDtypeStruct((6 * NC * NS, N), jnp.int32),
        scratch_types=[
            pltpu.VMEM((N,), jnp.int32),
            pltpu.VMEM((CHUNK * L,), jnp.int32),
        ])(_deg_kernel)
    return k(e0, e1, e2)


def _reduce_body(d_ref, s_ref):
    i = pl.program_id(0)
    part = jnp.sum(d_ref[...], axis=1).astype(jnp.float32)

    @pl.when(i == 0)
    def _():
        s_ref[...] = part

    @pl.when(i > 0)
    def _():
        s_ref[...] += part

    @pl.when(i == pl.num_programs(0) - 1)
    def _():
        s_ref[...] = lax.rsqrt(jnp.maximum(s_ref[...], 1.0))


def _scales(dcnt):
    stp = 8
    return pl.pallas_call(
        _reduce_body,
        grid=(NC * NS // stp,),
        in_specs=[pl.BlockSpec((6, stp, N), lambda i: (0, i, 0))],
        out_specs=pl.BlockSpec((6, N), lambda i: (0, 0)),
        out_shape=jax.ShapeDtypeStruct((6, N), jnp.float32),
    )(dcnt)


def _mm_body(x_ref, s_ref, w_ref, y_ref):
    xs = x_ref[...] * s_ref[0]
    y_ref[...] = jnp.dot(xs, w_ref[0],
                         preferred_element_type=jnp.float32)[None]


def _matmul(x, s_out, w_stack):
    blk = 2000
    return pl.pallas_call(
        _mm_body,
        grid=(3, N // blk),
        in_specs=[
            pl.BlockSpec((blk, D), lambda r, i: (i, 0)),
            pl.BlockSpec((1, blk, 1), lambda r, i: (r, i, 0)),
            pl.BlockSpec((1, D, D), lambda r, i: (r, 0, 0)),
        ],
        out_specs=pl.BlockSpec((1, blk, D), lambda r, i: (r, i, 0)),
        out_shape=jax.ShapeDtypeStruct((3, N, D), jnp.float32),
    )(x, s_out, w_stack)


def _make_agg_kernel(relu):
    def body(y, e0, e1, e2, s_in, bsum, out,
             acc0, acc1, acc2,
             stage_s, stage_d, pend_s, pend_l, rowbuf,
             abuf0, abuf1, abuf2, obuf, sbuf, bbuf):
        c = lax.axis_index("c")
        s = lax.axis_index("s")
        accs = (acc0, acc1, acc2)
        abufs = (abuf0, abuf1, abuf2)
        io = lax.iota(jnp.int32, L)

        pltpu.sync_copy(bsum, bbuf)

        nzero = -(-(BK // G) // NS)
        nchunks = -(-UPT16 // CHUNK)
        nunits_ep = ACC_ROWS - 128
        nep = -(-(BK // L) // NS)

        def bucket_body(ib, _):
            b = ib * NC + c
            base = b * BK
            def zrow(i, _):
                for q in range(D // L):
                    rowbuf[i, pl.ds(q * L, L)] = jnp.zeros((L,), jnp.float32)
                return 0
            lax.fori_loop(0, G, zrow, 0)
            for acc in accs:
                def zb(k2, _):
                    ci = s + NS * k2

                    @pl.when(ci < BK // G)
                    def _():
                        pltpu.sync_copy(rowbuf, acc.at[pl.ds(ci * G, G)])
                    return 0
                lax.fori_loop(0, nzero, zb, 0)
            plsc.subcore_barrier()

            lo = s * UPT16
            hi = jnp.minimum(lo + UPT16, U)
            for r, (acc, e) in enumerate(zip(accs, (e0, e1, e2))):
                yr = y.at[r]

                def fire(cnt):
                    pltpu.sync_copy(yr.at[pend_s.at[0]], rowbuf)
                    pltpu.sync_copy(rowbuf, acc.at[pend_l.at[0]], add=True)
                    return cnt

                def unit_body(v, cnt, sb):
                    uid = sb + v
                    act = (uid >= lo) & (uid < hi)
                    svec = stage_s[pl.ds(v * L, L)]
                    dvec = stage_d[pl.ds(v * L, L)]
                    local = dvec - base
                    m = (local >= 0) & (local < BK) & act
                    ones = jnp.where(m, 1, 0).astype(jnp.int32)
                    cs = plsc.cumsum(ones)
                    na = jnp.sum(ones)
                    pos = cnt + cs - 1
                    plsc.store_scatter(pend_s, [pos >> 7, pos & 127],
                                       svec, mask=m)
                    plsc.store_scatter(pend_l, [pos >> 7, pos & 127],
                                       local, mask=m)
                    cnt = cnt + na

                    def do_fire(cn):
                        cn = fire(cn)
                        pend_s[0, pl.ds(0, L)] = pend_s[1, pl.ds(0, L)]
                        pend_l[0, pl.ds(0, L)] = pend_l[1, pl.ds(0, L)]
                        return cn - G
                    return lax.cond(cnt >= G, do_fire, lambda cn: cn, cnt)

                def chunk_body(j, cnt):
                    ubase = lo + j * CHUNK

                    def process(cn):
                        sb = jnp.minimum(ubase, U - CHUNK)
                        pltpu.sync_copy(
                            e.at[pl.ds(sb * L, CHUNK * L)], stage_s)
                        pltpu.sync_copy(
                            e.at[pl.ds(E + sb * L, CHUNK * L)], stage_d)
                        nvalid = jnp.minimum(hi, ubase + CHUNK) - ubase
                        skip = ubase - sb
                        def ub(v, c2):
                            return lax.cond(
                                (v >= skip) & (v < skip + nvalid),
                                lambda c3: unit_body(v, c3, sb),
                                lambda c3: c3, c2)
                        return lax.fori_loop(0, CHUNK, ub, cn)
                    cnt = lax.cond(ubase < hi, process, lambda cn: cn, cnt)
                    return cnt
                cnt = lax.fori_loop(0, nchunks, chunk_body, jnp.int32(0))

                for q in range(G // L):
                    lid = io + q * L
                    keep = lid < cnt
                    cur_l = pend_l[0, pl.ds(q * L, L)]
                    pend_l[0, pl.ds(q * L, L)] = jnp.where(
                        keep, cur_l, jnp.full((L,), TRASH, jnp.int32))
                    cur_s = pend_s[0, pl.ds(q * L, L)]
                    pend_s[0, pl.ds(q * L, L)] = jnp.where(
                        keep, cur_s, jnp.zeros((L,), jnp.int32))

                @pl.when(cnt > 0)
                def _():
                    fire(jnp.int32(0))
            plsc.subcore_barrier()

            def ep_body(k2, _):
                u = s + NS * k2
                gbase = base + u * L

                @pl.when((u < BK // L) & (gbase < N))
                def _():
                    for r in range(3):
                        pltpu.sync_copy(accs[r].at[pl.ds(u * L, L)],
                                        abufs[r])
                        pltpu.sync_copy(s_in.at[r, pl.ds(gbase, L)],
                                        sbuf.at[r])

                    srow0 = sbuf[0, pl.ds(0, L)]
                    srow1 = sbuf[1, pl.ds(0, L)]
                    srow2 = sbuf[2, pl.ds(0, L)]

                    def row_body(i, _):
                        iv = jnp.zeros((L,), jnp.int32) + i
                        a0 = _vtake(srow0, iv)
                        a1 = _vtake(srow1, iv)
                        a2 = _vtake(srow2, iv)
                        for q in range(D // L):
                            sl = pl.ds(q * L, L)
                            v = (abuf0[i, sl] * a0 + abuf1[i, sl] * a1
                                 + abuf2[i, sl] * a2 + bbuf[sl])
                            if relu:
                                v = jnp.maximum(v, 0.0)
                            obuf[i, sl] = v
                        return 0
                    lax.fori_loop(0, L, row_body, 0)
                    pltpu.sync_copy(obuf, out.at[pl.ds(gbase, L)])
                return 0
            lax.fori_loop(0, nep, ep_body, 0)
            plsc.subcore_barrier()
            return 0
        lax.fori_loop(0, NBK // NC, bucket_body, 0)
    return body


def _aggregate(y, e0, e1, e2, s_in, bsum, relu):
    k = functools.partial(
        pl.kernel, mesh=_get_mesh(),
        compiler_params=pltpu.CompilerParams(needs_layout_passes=False),
        out_type=jax.ShapeDtypeStruct((N, D), jnp.float32),
        scratch_types=[
            pltpu.VMEM_SHARED((ACC_ROWS, D), jnp.float32),
            pltpu.VMEM_SHARED((ACC_ROWS, D), jnp.float32),
            pltpu.VMEM_SHARED((ACC_ROWS, D), jnp.float32),
            pltpu.VMEM((CHUNK * L,), jnp.int32),
            pltpu.VMEM((CHUNK * L,), jnp.int32),
            pltpu.VMEM((2, G), jnp.int32),
            pltpu.VMEM((2, G), jnp.int32),
            pltpu.VMEM((G, D), jnp.float32),
            pltpu.VMEM((L, D), jnp.float32),
            pltpu.VMEM((L, D), jnp.float32),
            pltpu.VMEM((L, D), jnp.float32),
            pltpu.VMEM((L, D), jnp.float32),
            pltpu.VMEM((3, L), jnp.float32),
            pltpu.VMEM((D,), jnp.float32),
        ])(_make_agg_kernel(relu))
    return k(y, e0, e1, e2, s_in, bsum)


def kernel(x, edge_index_r0, edge_index_r1, edge_index_r2,
           W1_r0, b1_r0, W1_r1, b1_r1, W1_r2, b1_r2,
           W2_r0, b2_r0, W2_r1, b2_r1, W2_r2, b2_r2):
    e0 = edge_index_r0.reshape(-1)
    e1 = edge_index_r1.reshape(-1)
    e2 = edge_index_r2.reshape(-1)
    dcnt = _degrees(e0, e1, e2)
    sgl = _scales(dcnt.reshape(6, NC * NS, N))
    sgl = sgl.reshape(3, 2, N)
    s_out = sgl[:, 0][..., None]
    s_in = sgl[:, 1]
    w1 = jnp.stack([W1_r0, W1_r1, W1_r2])
    w2 = jnp.stack([W2_r0, W2_r1, W2_r2])
    bsum1 = b1_r0 + b1_r1 + b1_r2
    bsum2 = b2_r0 + b2_r1 + b2_r2
    y1 = _matmul(x, s_out, w1)
    h = _aggregate(y1, e0, e1, e2, s_in, bsum1, relu=True)
    y2 = _matmul(h, s_out, w2)
    out = _aggregate(y2, e0, e1, e2, s_in, bsum2, relu=False)
    return out


# speedup vs baseline: 1.0238x; 1.0238x over previous
import functools

import jax
import jax.numpy as jnp
from jax import lax
from jax.experimental import pallas as pl
from jax.experimental.pallas import tpu as pltpu
from jax.experimental.pallas import tpu_sc as plsc

N = 50000
E = 200000
D = 128
NC = 2
NS = 16
L = 16

U = E // L
UPT32 = -(-U // (NC * NS))
UPT16 = -(-U // NS)
CHUNK = 200
BK = 3840
NBK = 14
ACC_ROWS = BK + 128
TRASH = BK
G = 128

_mesh = None


def _get_mesh():
    global _mesh
    if _mesh is None:
        _mesh = plsc.VectorSubcoreMesh(
            core_axis_name="c", subcore_axis_name="s",
            num_cores=NC, num_subcores=NS)
    return _mesh


_GDN = lax.GatherDimensionNumbers(
    offset_dims=(), collapsed_slice_dims=(0,), start_index_map=(0,))


def _vtake(x, i):
    return lax.gather(x, i[:, None], _GDN, slice_sizes=(1,),
                      mode=lax.GatherScatterMode.PROMISE_IN_BOUNDS)


def _dedup_add(cbuf, idx):
    cnt, last = plsc.scan_count(idx)
    plsc.addupdate_scatter(cbuf, [idx], cnt, mask=last)


def _deg_kernel(e0, e1, e2, dcnt, cbuf, stage):
    wid = lax.axis_index("s") * NC + lax.axis_index("c")
    lo = jnp.minimum(wid * UPT32, U)
    hi = jnp.minimum(lo + UPT32, U)
    nchunks = -(-UPT32 // CHUNK)
    for r, e in enumerate((e0, e1, e2)):
        for side in range(2):
            rs = r * 2 + side

            def zero_body(i, _):
                cbuf[pl.ds(i * L, L)] = jnp.zeros((L,), jnp.int32)
                return 0
            lax.fori_loop(0, N // L, zero_body, 0)

            def chunk_body(j, _):
                ubase = lo + j * CHUNK
                sb = jnp.minimum(ubase, U - CHUNK)
                pltpu.sync_copy(
                    e.at[pl.ds(side * E + sb * L, CHUNK * L)], stage)

                def unit_body(v, _):
                    uid = sb + v

                    @pl.when((uid >= ubase) & (uid < hi))
                    def _():
                        _dedup_add(cbuf, stage[pl.ds(v * L, L)])
                    return 0
                lax.fori_loop(0, CHUNK, unit_body, 0)
                return 0
            lax.fori_loop(0, nchunks, chunk_body, 0)
            pltpu.sync_copy(cbuf, dcnt.at[rs * (NC * NS) + wid])


def _degrees(e0, e1, e2):
    k = functools.partial(
        pl.kernel, mesh=_get_mesh(),
        compiler_params=pltpu.CompilerParams(needs_layout_passes=False),
        out_type=jax.ShapeDtypeStruct((6 * NC * NS, N), jnp.int32),
        scratch_types=[
            pltpu.VMEM((N,), jnp.int32),
            pltpu.VMEM((CHUNK * L,), jnp.int32),
        ])(_deg_kernel)
    return k(e0, e1, e2)


def _reduce_body(d_ref, s_ref):
    i = pl.program_id(0)
    part = jnp.sum(d_ref[...], axis=1).astype(jnp.float32)

    @pl.when(i == 0)
    def _():
        s_ref[...] = part

    @pl.when(i > 0)
    def _():
        s_ref[...] += part

    @pl.when(i == pl.num_programs(0) - 1)
    def _():
        s_ref[...] = lax.rsqrt(jnp.maximum(s_ref[...], 1.0))


def _scales(dcnt):
    stp = 8
    return pl.pallas_call(
        _reduce_body,
        grid=(NC * NS // stp,),
        in_specs=[pl.BlockSpec((6, stp, N), lambda i: (0, i, 0))],
        out_specs=pl.BlockSpec((6, N), lambda i: (0, 0)),
        out_shape=jax.ShapeDtypeStruct((6, N), jnp.float32),
    )(dcnt)


def _mm_body(x_ref, s_ref, w_ref, y_ref):
    xs = x_ref[...] * s_ref[0]
    y_ref[...] = jnp.dot(xs, w_ref[0],
                         preferred_element_type=jnp.float32)[None]


def _matmul(x, s_out, w_stack):
    blk = 2000
    return pl.pallas_call(
        _mm_body,
        grid=(3, N // blk),
        in_specs=[
            pl.BlockSpec((blk, D), lambda r, i: (i, 0)),
            pl.BlockSpec((1, blk, 1), lambda r, i: (r, i, 0)),
            pl.BlockSpec((1, D, D), lambda r, i: (r, 0, 0)),
        ],
        out_specs=pl.BlockSpec((1, blk, D), lambda r, i: (r, i, 0)),
        out_shape=jax.ShapeDtypeStruct((3, N, D), jnp.float32),
    )(x, s_out, w_stack)


def _make_agg_kernel(relu):
    def body(y, e0, e1, e2, s_in, bsum, out,
             acc0, acc1, acc2,
             stage_s, stage_d, pend_s, pend_l, rowbuf,
             abuf0, abuf1, abuf2, obuf, sbuf, bbuf):
        c = lax.axis_index("c")
        s = lax.axis_index("s")
        accs = (acc0, acc1, acc2)
        abufs = (abuf0, abuf1, abuf2)
        io = lax.iota(jnp.int32, L)

        pltpu.sync_copy(bsum, bbuf)

        nzero = -(-(BK // G) // NS)
        nchunks = -(-UPT16 // CHUNK)
        nunits_ep = ACC_ROWS - 128
        nep = -(-(BK // L) // NS)

        def bucket_body(ib, _):
            b = ib * NC + c
            base = b * BK
            def zrow(i, _):
                for q in range(D // L):
                    rowbuf[i, pl.ds(q * L, L)] = jnp.zeros((L,), jnp.float32)
                return 0
            lax.fori_loop(0, G, zrow, 0)
            for acc in accs:
                def zb(k2, _):
                    ci = s + NS * k2

                    @pl.when(ci < BK // G)
                    def _():
                        pltpu.sync_copy(rowbuf, acc.at[pl.ds(ci * G, G)])
                    return 0
                lax.fori_loop(0, nzero, zb, 0)
            plsc.subcore_barrier()

            lo = s * UPT16
            hi = jnp.minimum(lo + UPT16, U)
            for r, (acc, e) in enumerate(zip(accs, (e0, e1, e2))):
                yr = y.at[r]

                def fire(cnt):
                    pltpu.sync_copy(yr.at[pend_s.at[0]], rowbuf)
                    pltpu.sync_copy(rowbuf, acc.at[pend_l.at[0]], add=True)
                    return cnt

                def unit_body(v, cnt, sb):
                    uid = sb + v
                    act = (uid >= lo) & (uid < hi)
                    svec = stage_s[pl.ds(v * L, L)]
                    dvec = stage_d[pl.ds(v * L, L)]
                    local = dvec - base
                    m = (local >= 0) & (local < BK) & act
                    ones = jnp.where(m, 1, 0).astype(jnp.int32)
                    cs = plsc.cumsum(ones)
                    na = jnp.sum(ones)
                    pos = cnt + cs - 1
                    plsc.store_scatter(pend_s, [pos >> 7, pos & 127],
                                       svec, mask=m)
                    plsc.store_scatter(pend_l, [pos >> 7, pos & 127],
                                       local, mask=m)
                    cnt = cnt + na

                    def do_fire(cn):
                        cn = fire(cn)
                        pend_s[0, pl.ds(0, L)] = pend_s[1, pl.ds(0, L)]
                        pend_l[0, pl.ds(0, L)] = pend_l[1, pl.ds(0, L)]
                        return cn - G
                    return lax.cond(cnt >= G, do_fire, lambda cn: cn, cnt)

                def chunk_body(j, cnt):
                    ubase = lo + j * CHUNK

                    def process(cn):
                        sb = jnp.minimum(ubase, U - CHUNK)
                        pltpu.sync_copy(
                            e.at[pl.ds(sb * L, CHUNK * L)], stage_s)
                        pltpu.sync_copy(
                            e.at[pl.ds(E + sb * L, CHUNK * L)], stage_d)
                        nvalid = jnp.minimum(hi, ubase + CHUNK) - ubase
                        skip = ubase - sb
                        def ub(v, c2):
                            return lax.cond(
                                (v >= skip) & (v < skip + nvalid),
                                lambda c3: unit_body(v, c3, sb),
                                lambda c3: c3, c2)
                        return lax.fori_loop(0, CHUNK, ub, cn)
                    cnt = lax.cond(ubase < hi, process, lambda cn: cn, cnt)
                    return cnt
                cnt = lax.fori_loop(0, nchunks, chunk_body, jnp.int32(0))

                for q in range(G // L):
                    lid = io + q * L
                    keep = lid < cnt
                    cur_l = pend_l[0, pl.ds(q * L, L)]
                    pend_l[0, pl.ds(q * L, L)] = jnp.where(
                        keep, cur_l, jnp.full((L,), TRASH, jnp.int32))
                    cur_s = pend_s[0, pl.ds(q * L, L)]
                    pend_s[0, pl.ds(q * L, L)] = jnp.where(
                        keep, cur_s, jnp.zeros((L,), jnp.int32))

                @pl.when(cnt > 0)
                def _():
                    fire(jnp.int32(0))
            plsc.subcore_barrier()

            def ep_body(k2, _):
                u = s + NS * k2
                gbase = base + u * L

                @pl.when((u < BK // L) & (gbase < N))
                def _():
                    for r in range(3):
                        pltpu.sync_copy(accs[r].at[pl.ds(u * L, L)],
                                        abufs[r])
                        pltpu.sync_copy(s_in.at[r, pl.ds(gbase, L)],
                                        sbuf.at[r])

                    srow0 = sbuf[0, pl.ds(0, L)]
                    srow1 = sbuf[1, pl.ds(0, L)]
                    srow2 = sbuf[2, pl.ds(0, L)]

                    def row_body(i, _):
                        iv = jnp.zeros((L,), jnp.int32) + i
                        a0 = _vtake(srow0, iv)
                        a1 = _vtake(srow1, iv)
                        a2 = _vtake(srow2, iv)
                        for q in range(D // L):
                            sl = pl.ds(q * L, L)
                            v = (abuf0[i, sl] * a0 + abuf1[i, sl] * a1
                                 + abuf2[i, sl] * a2 + bbuf[sl])
                            if relu:
                                v = jnp.maximum(v, 0.0)
                            obuf[i, sl] = v
                        return 0
                    lax.fori_loop(0, L, row_body, 0)
                    pltpu.sync_copy(obuf, out.at[pl.ds(gbase, L)])
                return 0
            lax.fori_loop(0, nep, ep_body, 0)
            plsc.subcore_barrier()
            return 0
        lax.fori_loop(0, NBK // NC, bucket_body, 0)
    return body


def _aggregate(y, e0, e1, e2, s_in, bsum, relu):
    k = functools.partial(
        pl.kernel, mesh=_get_mesh(),
        compiler_params=pltpu.CompilerParams(needs_layout_passes=False),
        out_type=jax.ShapeDtypeStruct((N, D), jnp.float32),
        scratch_types=[
            pltpu.VMEM_SHARED((ACC_ROWS, D), jnp.float32),
            pltpu.VMEM_SHARED((ACC_ROWS, D), jnp.float32),
            pltpu.VMEM_SHARED((ACC_ROWS, D), jnp.float32),
            pltpu.VMEM((CHUNK * L,), jnp.int32),
            pltpu.VMEM((CHUNK * L,), jnp.int32),
            pltpu.VMEM((2, G), jnp.int32),
            pltpu.VMEM((2, G), jnp.int32),
            pltpu.VMEM((G, D), jnp.float32),
            pltpu.VMEM((L, D), jnp.float32),
            pltpu.VMEM((L, D), jnp.float32),
            pltpu.VMEM((L, D), jnp.float32),
            pltpu.VMEM((L, D), jnp.float32),
            pltpu.VMEM((3, L), jnp.float32),
            pltpu.VMEM((D,), jnp.float32),
        ])(_make_agg_kernel(relu))
    return k(y, e0, e1, e2, s_in, bsum)


def kernel(x, edge_index_r0, edge_index_r1, edge_index_r2,
           W1_r0, b1_r0, W1_r1, b1_r1, W1_r2, b1_r2,
           W2_r0, b2_r0, W2_r1, b2_r1, W2_r2, b2_r2):
    e0 = edge_index_r0.reshape(-1)
    e1 = edge_index_r1.reshape(-1)
    e2 = edge_index_r2.reshape(-1)
    dcnt = _degrees(e0, e1, e2)
    sgl = _scales(dcnt.reshape(6, NC * NS, N))
    sgl = sgl.reshape(3, 2, N)
    s_out = sgl[:, 0][..., None]
    s_in = sgl[:, 1]
    w1 = jnp.stack([W1_r0, W1_r1, W1_r2])
    w2 = jnp.stack([W2_r0, W2_r1, W2_r2])
    bsum1 = b1_r0 + b1_r1 + b1_r2
    bsum2 = b2_r0 + b2_r1 + b2_r2
    y1 = _matmul(x, s_out, w1)
    h = _aggregate(y1, e0, e1, e2, s_in, bsum1, relu=True)
    y2 = _matmul(h, s_out, w2)
    out = _aggregate(y2, e0, e1, e2, s_in, bsum2, relu=False)
    return out


# speedup vs baseline: 1.0650x; 1.0402x over previous
import functools

import jax
import jax.numpy as jnp
from jax import lax
from jax.experimental import pallas as pl
from jax.experimental.pallas import tpu as pltpu
from jax.experimental.pallas import tpu_sc as plsc

N = 50000
E = 200000
D = 128
NC = 2
NS = 16
L = 16

U = E // L
UPT32 = -(-U // (NC * NS))
UPT16 = -(-U // NS)
CHUNK = 200
BK = 3840
NBK = 14
ACC_ROWS = BK + 128
TRASH = BK
G = 128

_mesh = None


def _get_mesh():
    global _mesh
    if _mesh is None:
        _mesh = plsc.VectorSubcoreMesh(
            core_axis_name="c", subcore_axis_name="s",
            num_cores=NC, num_subcores=NS)
    return _mesh


_GDN = lax.GatherDimensionNumbers(
    offset_dims=(), collapsed_slice_dims=(0,), start_index_map=(0,))


def _vtake(x, i):
    return lax.gather(x, i[:, None], _GDN, slice_sizes=(1,),
                      mode=lax.GatherScatterMode.PROMISE_IN_BOUNDS)


def _dedup_add(cbuf, idx):
    cnt, last = plsc.scan_count(idx)
    plsc.addupdate_scatter(cbuf, [idx], cnt, mask=last)


def _deg_kernel(e0, e1, e2, dcnt, cbuf, stage):
    wid = lax.axis_index("s") * NC + lax.axis_index("c")
    lo = jnp.minimum(wid * UPT32, U)
    hi = jnp.minimum(lo + UPT32, U)
    nchunks = -(-UPT32 // CHUNK)
    for r, e in enumerate((e0, e1, e2)):
        for side in range(2):
            rs = r * 2 + side

            def zero_body(i, _):
                cbuf[pl.ds(i * L, L)] = jnp.zeros((L,), jnp.int32)
                return 0
            lax.fori_loop(0, N // L, zero_body, 0)

            def chunk_body(j, _):
                ubase = lo + j * CHUNK
                sb = jnp.minimum(ubase, U - CHUNK)
                pltpu.sync_copy(
                    e.at[pl.ds(side * E + sb * L, CHUNK * L)], stage)

                def unit_body(v, _):
                    uid = sb + v

                    @pl.when((uid >= ubase) & (uid < hi))
                    def _():
                        _dedup_add(cbuf, stage[pl.ds(v * L, L)])
                    return 0
                lax.fori_loop(0, CHUNK, unit_body, 0)
                return 0
            lax.fori_loop(0, nchunks, chunk_body, 0)
            pltpu.sync_copy(cbuf, dcnt.at[rs * (NC * NS) + wid])


def _degrees(e0, e1, e2):
    k = functools.partial(
        pl.kernel, mesh=_get_mesh(),
        compiler_params=pltpu.CompilerParams(needs_layout_passes=False),
        out_type=jax.ShapeDtypeStruct((6 * NC * NS, N), jnp.int32),
        scratch_types=[
            pltpu.VMEM((N,), jnp.int32),
            pltpu.VMEM((CHUNK * L,), jnp.int32),
        ])(_deg_kernel)
    return k(e0, e1, e2)


def _reduce_body(d_ref, s_ref):
    i = pl.program_id(0)
    part = jnp.sum(d_ref[...], axis=1).astype(jnp.float32)

    @pl.when(i == 0)
    def _():
        s_ref[...] = part

    @pl.when(i > 0)
    def _():
        s_ref[...] += part

    @pl.when(i == pl.num_programs(0) - 1)
    def _():
        s_ref[...] = lax.rsqrt(jnp.maximum(s_ref[...], 1.0))


def _scales(dcnt):
    stp = 8
    return pl.pallas_call(
        _reduce_body,
        grid=(NC * NS // stp,),
        in_specs=[pl.BlockSpec((6, stp, N), lambda i: (0, i, 0))],
        out_specs=pl.BlockSpec((6, N), lambda i: (0, 0)),
        out_shape=jax.ShapeDtypeStruct((6, N), jnp.float32),
    )(dcnt)


def _mm_body(x_ref, s_ref, w_ref, y_ref):
    xs = x_ref[...] * s_ref[0]
    y_ref[...] = jnp.dot(xs, w_ref[0],
                         preferred_element_type=jnp.float32)[None]


def _matmul(x, s_out, w_stack):
    blk = 2000
    return pl.pallas_call(
        _mm_body,
        grid=(3, N // blk),
        in_specs=[
            pl.BlockSpec((blk, D), lambda r, i: (i, 0)),
            pl.BlockSpec((1, blk, 1), lambda r, i: (r, i, 0)),
            pl.BlockSpec((1, D, D), lambda r, i: (r, 0, 0)),
        ],
        out_specs=pl.BlockSpec((1, blk, D), lambda r, i: (r, i, 0)),
        out_shape=jax.ShapeDtypeStruct((3, N, D), jnp.float32),
    )(x, s_out, w_stack)


def _make_agg_kernel(relu):
    def body(y, e0, e1, e2, s_in, bsum, out,
             acc0, acc1, acc2,
             stage_s, stage_d, pend_s, pend_l, rowbuf,
             abuf0, abuf1, abuf2, obuf, sbuf, bbuf):
        c = lax.axis_index("c")
        s = lax.axis_index("s")
        accs = (acc0, acc1, acc2)
        abufs = (abuf0, abuf1, abuf2)
        io = lax.iota(jnp.int32, L)

        pltpu.sync_copy(bsum, bbuf)

        nzero = -(-(BK // G) // NS)
        nchunks = -(-UPT16 // CHUNK)
        nunits_ep = ACC_ROWS - 128
        nep = -(-(BK // L) // NS)

        def bucket_body(ib, _):
            b = ib * NC + c
            base = b * BK
            def zrow(i, _):
                for q in range(D // L):
                    rowbuf[i, pl.ds(q * L, L)] = jnp.zeros((L,), jnp.float32)
                return 0
            lax.fori_loop(0, G, zrow, 0)
            for acc in accs:
                def zb(k2, _):
                    ci = s + NS * k2

                    @pl.when(ci < BK // G)
                    def _():
                        pltpu.sync_copy(rowbuf, acc.at[pl.ds(ci * G, G)])
                    return 0
                lax.fori_loop(0, nzero, zb, 0)
            plsc.subcore_barrier()

            lo = s * UPT16
            hi = jnp.minimum(lo + UPT16, U)
            for r, (acc, e) in enumerate(zip(accs, (e0, e1, e2))):
                yr = y.at[r]

                def fire(cnt):
                    pltpu.sync_copy(yr.at[pend_s.at[0]], rowbuf)
                    pltpu.sync_copy(rowbuf, acc.at[pend_l.at[0]], add=True)
                    return cnt

                def unit_body(v, cnt, sb, ubase):
                    uid = sb + v
                    act = (uid >= ubase) & (uid < jnp.minimum(hi, ubase + CHUNK))
                    svec = stage_s[pl.ds(v * L, L)]
                    dvec = stage_d[pl.ds(v * L, L)]
                    local = dvec - base
                    m = (local >= 0) & (local < BK) & act
                    ones = jnp.where(m, 1, 0).astype(jnp.int32)
                    cs = plsc.cumsum(ones)
                    na = jnp.sum(ones)
                    pos = cnt + cs - 1
                    plsc.store_scatter(pend_s, [pos >> 7, pos & 127],
                                       svec, mask=m)
                    plsc.store_scatter(pend_l, [pos >> 7, pos & 127],
                                       local, mask=m)
                    cnt = cnt + na

                    def do_fire(cn):
                        cn = fire(cn)
                        pend_s[0, pl.ds(0, L)] = pend_s[1, pl.ds(0, L)]
                        pend_l[0, pl.ds(0, L)] = pend_l[1, pl.ds(0, L)]
                        return cn - G
                    return lax.cond(cnt >= G, do_fire, lambda cn: cn, cnt)

                def chunk_body(j, cnt):
                    ubase = lo + j * CHUNK

                    def process(cn):
                        sb = jnp.minimum(ubase, U - CHUNK)
                        pltpu.sync_copy(
                            e.at[pl.ds(sb * L, CHUNK * L)], stage_s)
                        pltpu.sync_copy(
                            e.at[pl.ds(E + sb * L, CHUNK * L)], stage_d)
                        def ub(v, c2):
                            return unit_body(v, c2, sb, ubase)
                        return lax.fori_loop(0, CHUNK, ub, cn)
                    cnt = lax.cond(ubase < hi, process, lambda cn: cn, cnt)
                    return cnt
                cnt = lax.fori_loop(0, nchunks, chunk_body, jnp.int32(0))

                for q in range(G // L):
                    lid = io + q * L
                    keep = lid < cnt
                    cur_l = pend_l[0, pl.ds(q * L, L)]
                    pend_l[0, pl.ds(q * L, L)] = jnp.where(
                        keep, cur_l, jnp.full((L,), TRASH, jnp.int32))
                    cur_s = pend_s[0, pl.ds(q * L, L)]
                    pend_s[0, pl.ds(q * L, L)] = jnp.where(
                        keep, cur_s, jnp.zeros((L,), jnp.int32))

                @pl.when(cnt > 0)
                def _():
                    fire(jnp.int32(0))
            plsc.subcore_barrier()

            def ep_body(k2, _):
                u = s + NS * k2
                gbase = base + u * L

                @pl.when((u < BK // L) & (gbase < N))
                def _():
                    for r in range(3):
                        pltpu.sync_copy(accs[r].at[pl.ds(u * L, L)],
                                        abufs[r])
                        pltpu.sync_copy(s_in.at[r, pl.ds(gbase, L)],
                                        sbuf.at[r])

                    srow0 = sbuf[0, pl.ds(0, L)]
                    srow1 = sbuf[1, pl.ds(0, L)]
                    srow2 = sbuf[2, pl.ds(0, L)]

                    def row_body(i, _):
                        iv = jnp.zeros((L,), jnp.int32) + i
                        a0 = _vtake(srow0, iv)
                        a1 = _vtake(srow1, iv)
                        a2 = _vtake(srow2, iv)
                        for q in range(D // L):
                            sl = pl.ds(q * L, L)
                            v = (abuf0[i, sl] * a0 + abuf1[i, sl] * a1
                                 + abuf2[i, sl] * a2 + bbuf[sl])
                            if relu:
                                v = jnp.maximum(v, 0.0)
                            obuf[i, sl] = v
                        return 0
                    lax.fori_loop(0, L, row_body, 0)
                    pltpu.sync_copy(obuf, out.at[pl.ds(gbase, L)])
                return 0
            lax.fori_loop(0, nep, ep_body, 0)
            plsc.subcore_barrier()
            return 0
        lax.fori_loop(0, NBK // NC, bucket_body, 0)
    return body


def _aggregate(y, e0, e1, e2, s_in, bsum, relu):
    k = functools.partial(
        pl.kernel, mesh=_get_mesh(),
        compiler_params=pltpu.CompilerParams(needs_layout_passes=False),
        out_type=jax.ShapeDtypeStruct((N, D), jnp.float32),
        scratch_types=[
            pltpu.VMEM_SHARED((ACC_ROWS, D), jnp.float32),
            pltpu.VMEM_SHARED((ACC_ROWS, D), jnp.float32),
            pltpu.VMEM_SHARED((ACC_ROWS, D), jnp.float32),
            pltpu.VMEM((CHUNK * L,), jnp.int32),
            pltpu.VMEM((CHUNK * L,), jnp.int32),
            pltpu.VMEM((2, G), jnp.int32),
            pltpu.VMEM((2, G), jnp.int32),
            pltpu.VMEM((G, D), jnp.float32),
            pltpu.VMEM((L, D), jnp.float32),
            pltpu.VMEM((L, D), jnp.float32),
            pltpu.VMEM((L, D), jnp.float32),
            pltpu.VMEM((L, D), jnp.float32),
            pltpu.VMEM((3, L), jnp.float32),
            pltpu.VMEM((D,), jnp.float32),
        ])(_make_agg_kernel(relu))
    return k(y, e0, e1, e2, s_in, bsum)


def kernel(x, edge_index_r0, edge_index_r1, edge_index_r2,
           W1_r0, b1_r0, W1_r1, b1_r1, W1_r2, b1_r2,
           W2_r0, b2_r0, W2_r1, b2_r1, W2_r2, b2_r2):
    e0 = edge_index_r0.reshape(-1)
    e1 = edge_index_r1.reshape(-1)
    e2 = edge_index_r2.reshape(-1)
    dcnt = _degrees(e0, e1, e2)
    sgl = _scales(dcnt.reshape(6, NC * NS, N))
    sgl = sgl.reshape(3, 2, N)
    s_out = sgl[:, 0][..., None]
    s_in = sgl[:, 1]
    w1 = jnp.stack([W1_r0, W1_r1, W1_r2])
    w2 = jnp.stack([W2_r0, W2_r1, W2_r2])
    bsum1 = b1_r0 + b1_r1 + b1_r2
    bsum2 = b2_r0 + b2_r1 + b2_r2
    y1 = _matmul(x, s_out, w1)
    h = _aggregate(y1, e0, e1, e2, s_in, bsum1, relu=True)
    y2 = _matmul(h, s_out, w2)
    out = _aggregate(y2, e0, e1, e2, s_in, bsum2, relu=False)
    return out
